# Initial kernel scaffold; baseline (speedup 1.0000x reference)
import jax
import jax.numpy as jnp
from jax.experimental import pallas as pl


def kernel(pred_mask, true_mask):
    raise NotImplementedError("write your pallas kernel here")


# speedup vs baseline: 2.1682x; 2.1682x over previous
import jax
import jax.numpy as jnp
from jax.experimental import pallas as pl
from jax.experimental.pallas import tpu as pltpu

K = 64
P = 1024 * 1024
NC = 2
NS = 16
B = P // (NC * NS)


def _hist_body(pm_ref, tm_ref, out_ref):
    s = pl.program_id(1)
    pm = pm_ref[0].astype(jnp.bfloat16)
    tm = tm_ref[0].astype(jnp.bfloat16)
    ids = jax.lax.broadcasted_iota(jnp.int32, (K, 1), 0).astype(jnp.bfloat16)
    one = jnp.bfloat16(1.0)
    zero = jnp.bfloat16(0.0)
    p_oh = jnp.where(pm == ids, one, zero)
    t_oh = jnp.where(tm == ids, one, zero)
    part = jax.lax.dot_general(
        p_oh, t_oh, (((1,), (1,)), ((), ())),
        preferred_element_type=jnp.float32)

    @pl.when(s == 0)
    def _init():
        out_ref[0] = part

    @pl.when(s > 0)
    def _acc():
        out_ref[0] += part


def _loss_body(hist_ref, out_ref):
    inter = hist_ref[0] + hist_ref[1]
    area_p = jnp.sum(inter, axis=1, keepdims=True)
    area_t = jnp.sum(inter, axis=0, keepdims=True)
    union = area_p + area_t - inter
    iou = jnp.where(union > 0, inter / jnp.maximum(union, 1.0), 0.0)

    col = jax.lax.broadcasted_iota(jnp.int32, (K, K), 1)
    row = jax.lax.broadcasted_iota(jnp.int32, (K, K), 0)
    iou_c = jnp.where(col == 0, 0.0, iou)
    iou_r = jnp.where(row == 0, 0.0, iou)

    max_p = jnp.max(iou_c, axis=1, keepdims=True)
    max_t = jnp.max(iou_r, axis=0, keepdims=True)

    rid = jax.lax.broadcasted_iota(jnp.int32, (K, 1), 0)
    cid = jax.lax.broadcasted_iota(jnp.int32, (1, K), 1)
    pres_p = (area_p > 0) & (rid > 0)
    pres_t = (area_t > 0) & (cid > 0)

    loss = (jnp.sum(jnp.where(pres_p, 1.0 - max_p, 0.0), axis=0, keepdims=True)
            + jnp.sum(jnp.where(pres_t, 1.0 - max_t, 0.0), axis=1, keepdims=True))
    n = (jnp.sum(pres_p.astype(jnp.float32), axis=0, keepdims=True)
         + jnp.sum(pres_t.astype(jnp.float32), axis=1, keepdims=True))
    out_ref[...] = jnp.where(n > 0, loss / jnp.maximum(n, 1.0), 0.0)


def kernel(pred_mask, true_mask):
    nblk = NC * NS
    pm = pred_mask.reshape(nblk, 1, B)
    tm = true_mask.reshape(nblk, 1, B)

    hist = pl.pallas_call(
        _hist_body,
        out_shape=jax.ShapeDtypeStruct((NC, K, K), jnp.float32),
        grid=(NC, NS),
        in_specs=[
            pl.BlockSpec((1, 1, B), lambda c, s: (c * NS + s, 0, 0)),
            pl.BlockSpec((1, 1, B), lambda c, s: (c * NS + s, 0, 0)),
        ],
        out_specs=pl.BlockSpec((1, K, K), lambda c, s: (c, 0, 0)),
        compiler_params=pltpu.CompilerParams(
            dimension_semantics=("parallel", "arbitrary"),
        ),
        name="iou_hist",
    )(pm, tm)

    loss = pl.pallas_call(
        _loss_body,
        out_shape=jax.ShapeDtypeStruct((1, 1), jnp.float32),
        name="iou_loss",
    )(hist)
    return loss[0, 0]
